# speedup vs baseline: 7.2522x; 7.2522x over previous
import functools

import jax
import jax.numpy as jnp
from jax import lax
from jax.experimental import pallas as pl
from jax.experimental.pallas import tpu as pltpu
from jax.experimental.pallas import tpu_sc as plsc

N = 10000
D = 128
E = 320000
NP = 10240
NTILES = 32
CH = 128
NCH = 80
EPT = NCH * CH
EPAD = NTILES * EPT
ROWS_PER_TILE = NP // 16


def _mesh():
    return plsc.VectorSubcoreMesh(core_axis_name="c", subcore_axis_name="s")


def _sc_deg(dst_blk):
    @functools.partial(
        pl.kernel,
        mesh=_mesh(),
        out_type=jax.ShapeDtypeStruct((2, NP), jnp.float32),
        scratch_types=[
            pltpu.VMEM((NCH, CH), jnp.int32),
            pltpu.VMEM((CH,), jnp.float32),
            pltpu.VMEM((ROWS_PER_TILE,), jnp.float32),
            pltpu.VMEM_SHARED((NP,), jnp.float32),
        ],
    )
    def k(dst_hbm, out_hbm, dstv, onesv, zerosv, deg_acc):
        c = lax.axis_index("c")
        s = lax.axis_index("s")
        wid = s * 2 + c
        for i in range(CH // 16):
            onesv[pl.ds(i * 16, 16)] = jnp.ones((16,), jnp.float32)
        for i in range(ROWS_PER_TILE // 16):
            zerosv[pl.ds(i * 16, 16)] = jnp.zeros((16,), jnp.float32)
        base = s * ROWS_PER_TILE
        pltpu.sync_copy(zerosv, deg_acc.at[pl.ds(base, ROWS_PER_TILE)])
        pltpu.sync_copy(dst_hbm.at[wid], dstv)
        plsc.subcore_barrier()

        def step(j, carry):
            pltpu.sync_copy(onesv, deg_acc.at[dstv.at[j]], add=True)
            return carry

        lax.fori_loop(0, NCH, step, 0)
        plsc.subcore_barrier()
        pltpu.sync_copy(deg_acc.at[pl.ds(base, ROWS_PER_TILE)],
                        out_hbm.at[c, pl.ds(base, ROWS_PER_TILE)])

    return k(dst_blk)


def _sc_agg(h, src_blk, dst_blk):
    @functools.partial(
        pl.kernel,
        mesh=_mesh(),
        out_type=jax.ShapeDtypeStruct((2, NP, D), jnp.float32),
        scratch_types=[
            pltpu.VMEM((NCH, CH), jnp.int32),
            pltpu.VMEM((NCH, CH), jnp.int32),
            pltpu.VMEM((CH, D), jnp.float32),
            pltpu.VMEM_SHARED((NP, D), jnp.float32),
            pltpu.SemaphoreType.DMA,
        ],
    )
    def k(h_hbm, src_hbm, dst_hbm, out_hbm, srcv, dstv, rows, acc, sem):
        c = lax.axis_index("c")
        s = lax.axis_index("s")
        wid = s * 2 + c
        base = s * ROWS_PER_TILE
        pltpu.sync_copy(h_hbm.at[pl.ds(base, ROWS_PER_TILE)],
                        acc.at[pl.ds(base, ROWS_PER_TILE)])
        pltpu.sync_copy(src_hbm.at[wid], srcv)
        pltpu.sync_copy(dst_hbm.at[wid], dstv)
        plsc.subcore_barrier()

        def step(j, carry):
            pltpu.async_copy(h_hbm.at[srcv.at[j]], rows, sem).wait()
            pltpu.sync_copy(rows, acc.at[dstv.at[j]], add=True)
            return carry

        lax.fori_loop(0, NCH, step, 0)
        plsc.subcore_barrier()
        pltpu.sync_copy(acc.at[pl.ds(base, ROWS_PER_TILE)],
                        out_hbm.at[c, pl.ds(base, ROWS_PER_TILE)])

    return k(h, src_blk, dst_blk)


def _tc_dinv(degp2):
    def body(degp_ref, o_ref):
        o_ref[...] = lax.rsqrt(degp_ref[0] + degp_ref[1] + 1.0)

    return pl.pallas_call(
        body, out_shape=jax.ShapeDtypeStruct((NP // D, D), jnp.float32)
    )(degp2)


def _tc_pre(x_pad, W1, dinvf):
    def body(x_ref, w_ref, dinv_ref, o_ref):
        h = jnp.dot(x_ref[...], w_ref[...], preferred_element_type=jnp.float32)
        o_ref[...] = h * dinv_ref[...]

    return pl.pallas_call(
        body, out_shape=jax.ShapeDtypeStruct((NP, D), jnp.float32)
    )(x_pad, W1, dinvf)


def _tc_mid(p, h_prev, dinvf, b, g, be, W_next):
    def body(p_ref, h_ref, dinv_ref, b_ref, g_ref, be_ref, w_ref, o_ref):
        z = (p_ref[0] + p_ref[1] - h_ref[...]) * dinv_ref[...] + b_ref[...]
        rid = lax.broadcasted_iota(jnp.int32, (NP, D), 0)
        valid = rid < N
        zm = jnp.where(valid, z, 0.0)
        mean = jnp.sum(zm, axis=0, keepdims=True) * (1.0 / N)
        d0 = jnp.where(valid, z - mean, 0.0)
        var = jnp.sum(d0 * d0, axis=0, keepdims=True) * (1.0 / N)
        y = (z - mean) * lax.rsqrt(var + 1e-5) * g_ref[...] + be_ref[...]
        y = jnp.where(valid, jnp.maximum(y, 0.0), 0.0)
        o_ref[...] = jnp.dot(
            y, w_ref[...], preferred_element_type=jnp.float32
        ) * dinv_ref[...]

    return pl.pallas_call(
        body, out_shape=jax.ShapeDtypeStruct((NP, D), jnp.float32)
    )(p, h_prev, dinvf, b, g, be, W_next)


def _tc_final(p, h_prev, dinvf, b):
    def body(p_ref, h_ref, dinv_ref, b_ref, o_ref):
        z = (p_ref[0] + p_ref[1] - h_ref[...]) * dinv_ref[...] + b_ref[...]
        m = jnp.max(z, axis=1, keepdims=True)
        e = jnp.exp(z - m)
        o_ref[...] = e / jnp.sum(e, axis=1, keepdims=True)

    return pl.pallas_call(
        body, out_shape=jax.ShapeDtypeStruct((NP, D), jnp.float32)
    )(p, h_prev, dinvf, b)


def kernel(x, edge_index, W1, b1, g1, be1, W2, b2, g2, be2, W3, b3):
    src = edge_index[0].astype(jnp.int32)
    dst = edge_index[1].astype(jnp.int32)
    src_p = jnp.concatenate([src, jnp.zeros((EPAD - E,), jnp.int32)])
    dst_p = jnp.concatenate([dst, jnp.full((EPAD - E,), N, jnp.int32)])
    src_blk = src_p.reshape(NTILES, NCH, CH)
    dst_blk = dst_p.reshape(NTILES, NCH, CH)
    x_pad = jnp.pad(x, ((0, NP - N), (0, 0)))
    b1r, b2r, b3r = b1.reshape(1, D), b2.reshape(1, D), b3.reshape(1, D)
    g1r, g2r = g1.reshape(1, D), g2.reshape(1, D)
    be1r, be2r = be1.reshape(1, D), be2.reshape(1, D)

    degp = _sc_deg(dst_blk)
    dinv = _tc_dinv(degp.reshape(2, NP // D, D))
    dinvf = jnp.broadcast_to(dinv.reshape(NP, 1), (NP, D))

    h1 = _tc_pre(x_pad, W1, dinvf)
    p1 = _sc_agg(h1, src_blk, dst_blk)
    h2 = _tc_mid(p1, h1, dinvf, b1r, g1r, be1r, W2)
    p2 = _sc_agg(h2, src_blk, dst_blk)
    h3 = _tc_mid(p2, h2, dinvf, b2r, g2r, be2r, W3)
    p3 = _sc_agg(h3, src_blk, dst_blk)
    out = _tc_final(p3, h3, dinvf, b3r)
    return out[:N]


# speedup vs baseline: 7.3815x; 1.0178x over previous
import functools

import jax
import jax.numpy as jnp
from jax import lax
from jax.experimental import pallas as pl
from jax.experimental.pallas import tpu as pltpu
from jax.experimental.pallas import tpu_sc as plsc

N = 10000
D = 128
E = 320000
NP = 10240
NTILES = 32
CH = 128
NCH = 80
EPT = NCH * CH
EPAD = NTILES * EPT
ROWS_PER_TILE = NP // 16


def _mesh():
    return plsc.VectorSubcoreMesh(core_axis_name="c", subcore_axis_name="s")


def _sc_deg(dst_blk):
    @functools.partial(
        pl.kernel,
        mesh=_mesh(),
        out_type=jax.ShapeDtypeStruct((2, NP), jnp.float32),
        scratch_types=[
            pltpu.VMEM((NCH, CH), jnp.int32),
            pltpu.VMEM((CH,), jnp.float32),
            pltpu.VMEM((ROWS_PER_TILE,), jnp.float32),
            pltpu.VMEM_SHARED((NP,), jnp.float32),
        ],
    )
    def k(dst_hbm, out_hbm, dstv, onesv, zerosv, deg_acc):
        c = lax.axis_index("c")
        s = lax.axis_index("s")
        wid = s * 2 + c
        for i in range(CH // 16):
            onesv[pl.ds(i * 16, 16)] = jnp.ones((16,), jnp.float32)
        for i in range(ROWS_PER_TILE // 16):
            zerosv[pl.ds(i * 16, 16)] = jnp.zeros((16,), jnp.float32)
        base = s * ROWS_PER_TILE
        pltpu.sync_copy(zerosv, deg_acc.at[pl.ds(base, ROWS_PER_TILE)])
        pltpu.sync_copy(dst_hbm.at[wid], dstv)
        plsc.subcore_barrier()

        def step(j, carry):
            pltpu.sync_copy(onesv, deg_acc.at[dstv.at[j]], add=True)
            return carry

        lax.fori_loop(0, NCH, step, 0)
        plsc.subcore_barrier()
        pltpu.sync_copy(deg_acc.at[pl.ds(base, ROWS_PER_TILE)],
                        out_hbm.at[c, pl.ds(base, ROWS_PER_TILE)])

    return k(dst_blk)


def _sc_agg(h, src_blk, dst_blk):
    @functools.partial(
        pl.kernel,
        mesh=_mesh(),
        out_type=jax.ShapeDtypeStruct((2, NP, D), jnp.float32),
        scratch_types=[
            pltpu.VMEM((NCH // 2, CH), jnp.int32),
            pltpu.VMEM((NCH // 2, CH), jnp.int32),
            pltpu.VMEM((2, CH, D), jnp.float32),
            pltpu.VMEM_SHARED((NP, D), jnp.float32),
            pltpu.SemaphoreType.DMA,
            pltpu.SemaphoreType.DMA,
        ],
    )
    def k(h_hbm, src_hbm, dst_hbm, out_hbm, srcv, dstv, rows, acc, sem0, sem1):
        c = lax.axis_index("c")
        s = lax.axis_index("s")
        wid = s * 2 + c
        base = s * ROWS_PER_TILE
        pltpu.sync_copy(h_hbm.at[pl.ds(base, ROWS_PER_TILE)],
                        acc.at[pl.ds(base, ROWS_PER_TILE)])
        plsc.subcore_barrier()

        def step(i, carry):
            j = i * 2
            cp0 = pltpu.async_copy(h_hbm.at[srcv.at[j]], rows.at[0], sem0)
            cp1 = pltpu.async_copy(h_hbm.at[srcv.at[j + 1]], rows.at[1], sem1)
            cp0.wait()
            pltpu.sync_copy(rows.at[0], acc.at[dstv.at[j]], add=True)
            cp1.wait()
            pltpu.sync_copy(rows.at[1], acc.at[dstv.at[j + 1]], add=True)
            return carry

        for half in range(2):
            pltpu.sync_copy(src_hbm.at[wid, pl.ds(half * (NCH // 2), NCH // 2)],
                            srcv)
            pltpu.sync_copy(dst_hbm.at[wid, pl.ds(half * (NCH // 2), NCH // 2)],
                            dstv)
            lax.fori_loop(0, NCH // 4, step, 0)
        plsc.subcore_barrier()
        pltpu.sync_copy(acc.at[pl.ds(base, ROWS_PER_TILE)],
                        out_hbm.at[c, pl.ds(base, ROWS_PER_TILE)])

    return k(h, src_blk, dst_blk)


def _tc_dinv(degp2):
    def body(degp_ref, o_ref):
        o_ref[...] = lax.rsqrt(degp_ref[0] + degp_ref[1] + 1.0)

    return pl.pallas_call(
        body, out_shape=jax.ShapeDtypeStruct((NP // D, D), jnp.float32)
    )(degp2)


def _tc_pre(x_pad, W1, dinvf):
    def body(x_ref, w_ref, dinv_ref, o_ref):
        h = jnp.dot(x_ref[...], w_ref[...], preferred_element_type=jnp.float32)
        o_ref[...] = h * dinv_ref[...]

    return pl.pallas_call(
        body, out_shape=jax.ShapeDtypeStruct((NP, D), jnp.float32)
    )(x_pad, W1, dinvf)


def _tc_mid(p, h_prev, dinvf, b, g, be, W_next):
    def body(p_ref, h_ref, dinv_ref, b_ref, g_ref, be_ref, w_ref, o_ref):
        z = (p_ref[0] + p_ref[1] - h_ref[...]) * dinv_ref[...] + b_ref[...]
        rid = lax.broadcasted_iota(jnp.int32, (NP, D), 0)
        valid = rid < N
        zm = jnp.where(valid, z, 0.0)
        mean = jnp.sum(zm, axis=0, keepdims=True) * (1.0 / N)
        d0 = jnp.where(valid, z - mean, 0.0)
        var = jnp.sum(d0 * d0, axis=0, keepdims=True) * (1.0 / N)
        y = (z - mean) * lax.rsqrt(var + 1e-5) * g_ref[...] + be_ref[...]
        y = jnp.where(valid, jnp.maximum(y, 0.0), 0.0)
        o_ref[...] = jnp.dot(
            y, w_ref[...], preferred_element_type=jnp.float32
        ) * dinv_ref[...]

    return pl.pallas_call(
        body, out_shape=jax.ShapeDtypeStruct((NP, D), jnp.float32)
    )(p, h_prev, dinvf, b, g, be, W_next)


def _tc_final(p, h_prev, dinvf, b):
    def body(p_ref, h_ref, dinv_ref, b_ref, o_ref):
        z = (p_ref[0] + p_ref[1] - h_ref[...]) * dinv_ref[...] + b_ref[...]
        m = jnp.max(z, axis=1, keepdims=True)
        e = jnp.exp(z - m)
        o_ref[...] = e / jnp.sum(e, axis=1, keepdims=True)

    return pl.pallas_call(
        body, out_shape=jax.ShapeDtypeStruct((NP, D), jnp.float32)
    )(p, h_prev, dinvf, b)


def kernel(x, edge_index, W1, b1, g1, be1, W2, b2, g2, be2, W3, b3):
    src = edge_index[0].astype(jnp.int32)
    dst = edge_index[1].astype(jnp.int32)
    src_p = jnp.concatenate([src, jnp.zeros((EPAD - E,), jnp.int32)])
    dst_p = jnp.concatenate([dst, jnp.full((EPAD - E,), N, jnp.int32)])
    src_blk = src_p.reshape(NTILES, NCH, CH)
    dst_blk = dst_p.reshape(NTILES, NCH, CH)
    x_pad = jnp.pad(x, ((0, NP - N), (0, 0)))
    b1r, b2r, b3r = b1.reshape(1, D), b2.reshape(1, D), b3.reshape(1, D)
    g1r, g2r = g1.reshape(1, D), g2.reshape(1, D)
    be1r, be2r = be1.reshape(1, D), be2.reshape(1, D)

    degp = _sc_deg(dst_blk)
    dinv = _tc_dinv(degp.reshape(2, NP // D, D))
    dinvf = jnp.broadcast_to(dinv.reshape(NP, 1), (NP, D))

    h1 = _tc_pre(x_pad, W1, dinvf)
    p1 = _sc_agg(h1, src_blk, dst_blk)
    h2 = _tc_mid(p1, h1, dinvf, b1r, g1r, be1r, W2)
    p2 = _sc_agg(h2, src_blk, dst_blk)
    h3 = _tc_mid(p2, h2, dinvf, b2r, g2r, be2r, W3)
    p3 = _sc_agg(h3, src_blk, dst_blk)
    out = _tc_final(p3, h3, dinvf, b3r)
    return out[:N]


# speedup vs baseline: 21.7970x; 2.9529x over previous
import functools

import jax
import jax.numpy as jnp
from jax import lax
from jax.experimental import pallas as pl
from jax.experimental.pallas import tpu as pltpu
from jax.experimental.pallas import tpu_sc as plsc

N = 10000
D = 128
E = 320000
NP = 10240
NTILES = 32
CH = 128
NCH = 80
EPT = NCH * CH
EPAD = NTILES * EPT
ROWS_PER_TILE = NP // 16


def _mesh():
    return plsc.VectorSubcoreMesh(core_axis_name="c", subcore_axis_name="s")


def _sc_deg(dst_blk):
    @functools.partial(
        pl.kernel,
        mesh=_mesh(),
        out_type=jax.ShapeDtypeStruct((2, NP), jnp.float32),
        scratch_types=[
            pltpu.VMEM((NCH, CH), jnp.int32),
            pltpu.VMEM((CH,), jnp.float32),
            pltpu.VMEM((ROWS_PER_TILE,), jnp.float32),
            pltpu.VMEM_SHARED((NP,), jnp.float32),
        ],
    )
    def k(dst_hbm, out_hbm, dstv, onesv, zerosv, deg_acc):
        c = lax.axis_index("c")
        s = lax.axis_index("s")
        wid = s * 2 + c
        for i in range(CH // 16):
            onesv[pl.ds(i * 16, 16)] = jnp.ones((16,), jnp.float32)
        for i in range(ROWS_PER_TILE // 16):
            zerosv[pl.ds(i * 16, 16)] = jnp.zeros((16,), jnp.float32)
        base = s * ROWS_PER_TILE
        pltpu.sync_copy(zerosv, deg_acc.at[pl.ds(base, ROWS_PER_TILE)])
        pltpu.sync_copy(dst_hbm.at[wid], dstv)
        plsc.subcore_barrier()

        def step(j, carry):
            pltpu.sync_copy(onesv, deg_acc.at[dstv.at[j]], add=True)
            return carry

        lax.fori_loop(0, NCH, step, 0)
        plsc.subcore_barrier()
        pltpu.sync_copy(deg_acc.at[pl.ds(base, ROWS_PER_TILE)],
                        out_hbm.at[c, pl.ds(base, ROWS_PER_TILE)])

    return k(dst_blk)


def _sc_agg(h, src_blk, dst_blk):
    @functools.partial(
        pl.kernel,
        mesh=_mesh(),
        out_type=jax.ShapeDtypeStruct((2, NP, D), jnp.float32),
        scratch_types=[
            pltpu.VMEM((NCH // 2, CH), jnp.int32),
            pltpu.VMEM((NCH // 2, CH), jnp.int32),
            pltpu.VMEM((2, CH, D), jnp.float32),
            pltpu.VMEM_SHARED((NP, D), jnp.float32),
            pltpu.SemaphoreType.DMA,
            pltpu.SemaphoreType.DMA,
        ],
    )
    def k(h_hbm, src_hbm, dst_hbm, out_hbm, srcv, dstv, rows, acc, sem0, sem1):
        c = lax.axis_index("c")
        s = lax.axis_index("s")
        wid = s * 2 + c
        base = s * ROWS_PER_TILE
        pltpu.sync_copy(h_hbm.at[pl.ds(base, ROWS_PER_TILE)],
                        acc.at[pl.ds(base, ROWS_PER_TILE)])
        plsc.subcore_barrier()

        def step(i, carry):
            j = i * 2
            cp0 = pltpu.async_copy(h_hbm.at[srcv.at[j]], rows.at[0], sem0)
            cp1 = pltpu.async_copy(h_hbm.at[srcv.at[j + 1]], rows.at[1], sem1)
            cp0.wait()
            pltpu.sync_copy(rows.at[0], acc.at[dstv.at[j]], add=True)
            cp1.wait()
            pltpu.sync_copy(rows.at[1], acc.at[dstv.at[j + 1]], add=True)
            return carry

        for half in range(2):
            pltpu.sync_copy(src_hbm.at[wid, pl.ds(half * (NCH // 2), NCH // 2)],
                            srcv)
            pltpu.sync_copy(dst_hbm.at[wid, pl.ds(half * (NCH // 2), NCH // 2)],
                            dstv)
            lax.fori_loop(0, NCH // 4, step, 0)
        plsc.subcore_barrier()
        pltpu.sync_copy(acc.at[pl.ds(base, ROWS_PER_TILE)],
                        out_hbm.at[c, pl.ds(base, ROWS_PER_TILE)])

    return k(h, src_blk, dst_blk)


def _tc_dinv(degp2):
    def body(degp_ref, o_ref):
        o_ref[...] = lax.rsqrt(degp_ref[0] + degp_ref[1] + 1.0)

    return pl.pallas_call(
        body, out_shape=jax.ShapeDtypeStruct((NP // D, D), jnp.float32)
    )(degp2)


def _tc_pre(x_pad, W1, dinvf):
    def body(x_ref, w_ref, dinv_ref, o_ref):
        h = jnp.dot(x_ref[...], w_ref[...], preferred_element_type=jnp.float32)
        o_ref[...] = h * dinv_ref[...]

    return pl.pallas_call(
        body, out_shape=jax.ShapeDtypeStruct((NP, D), jnp.float32)
    )(x_pad, W1, dinvf)


def _tc_mid(p, h_prev, dinvf, b, g, be, W_next):
    def body(p_ref, h_ref, dinv_ref, b_ref, g_ref, be_ref, w_ref, o_ref):
        z = (p_ref[0] + p_ref[1] - h_ref[...]) * dinv_ref[...] + b_ref[...]
        rid = lax.broadcasted_iota(jnp.int32, (NP, D), 0)
        valid = rid < N
        zm = jnp.where(valid, z, 0.0)
        mean = jnp.sum(zm, axis=0, keepdims=True) * (1.0 / N)
        d0 = jnp.where(valid, z - mean, 0.0)
        var = jnp.sum(d0 * d0, axis=0, keepdims=True) * (1.0 / N)
        y = (z - mean) * lax.rsqrt(var + 1e-5) * g_ref[...] + be_ref[...]
        y = jnp.where(valid, jnp.maximum(y, 0.0), 0.0)
        o_ref[...] = jnp.dot(
            y, w_ref[...], preferred_element_type=jnp.float32
        ) * dinv_ref[...]

    return pl.pallas_call(
        body, out_shape=jax.ShapeDtypeStruct((NP, D), jnp.float32)
    )(p, h_prev, dinvf, b, g, be, W_next)


def _tc_final(p, h_prev, dinvf, b):
    def body(p_ref, h_ref, dinv_ref, b_ref, o_ref):
        z = (p_ref[0] + p_ref[1] - h_ref[...]) * dinv_ref[...] + b_ref[...]
        m = jnp.max(z, axis=1, keepdims=True)
        e = jnp.exp(z - m)
        o_ref[...] = e / jnp.sum(e, axis=1, keepdims=True)

    return pl.pallas_call(
        body, out_shape=jax.ShapeDtypeStruct((NP, D), jnp.float32)
    )(p, h_prev, dinvf, b)


def kernel(x, edge_index, W1, b1, g1, be1, W2, b2, g2, be2, W3, b3):
    src = edge_index[0].astype(jnp.int32)
    dst = edge_index[1].astype(jnp.int32)
    pad = jnp.arange(EPAD - E, dtype=jnp.int32)
    src_p = jnp.concatenate([src, pad % N])
    dst_p = jnp.concatenate([dst, N + pad % (NP - N)])
    src_blk = src_p.reshape(NTILES, NCH, CH)
    dst_blk = dst_p.reshape(NTILES, NCH, CH)
    x_pad = jnp.pad(x, ((0, NP - N), (0, 0)))
    b1r, b2r, b3r = b1.reshape(1, D), b2.reshape(1, D), b3.reshape(1, D)
    g1r, g2r = g1.reshape(1, D), g2.reshape(1, D)
    be1r, be2r = be1.reshape(1, D), be2.reshape(1, D)

    degp = _sc_deg(dst_blk)
    dinv = _tc_dinv(degp.reshape(2, NP // D, D))
    dinvf = jnp.broadcast_to(dinv.reshape(NP, 1), (NP, D))

    h1 = _tc_pre(x_pad, W1, dinvf)
    p1 = _sc_agg(h1, src_blk, dst_blk)
    h2 = _tc_mid(p1, h1, dinvf, b1r, g1r, be1r, W2)
    p2 = _sc_agg(h2, src_blk, dst_blk)
    h3 = _tc_mid(p2, h2, dinvf, b2r, g2r, be2r, W3)
    p3 = _sc_agg(h3, src_blk, dst_blk)
    out = _tc_final(p3, h3, dinvf, b3r)
    return out[:N]
